# Initial kernel scaffold; baseline (speedup 1.0000x reference)
import functools

import jax
import jax.numpy as jnp
from jax import lax
from jax.experimental import pallas as pl
from jax.experimental.pallas import tpu as pltpu
from jax.experimental.pallas import tpu_sc as plsc

N = 10000
E = 160000
IN_DIM = 1024
HID = 256
BN_SCALE = 1.0 / (1.0 + 1e-5) ** 0.5

_NSC = 2
_NTILE = 16
_K = 128
_NCH = E // _K
_ROWS_PER_TILE = N // _NTILE

_PREC = jax.lax.Precision.HIGHEST


def _dotT(a, w):
    return lax.dot_general(a, w, (((1,), (1,)), ((), ())),
                           preferred_element_type=jnp.float32,
                           precision=_PREC)


def _emb_body(x_ref, w_ref, b_ref, o_ref):
    o_ref[...] = _dotT(x_ref[...], w_ref[...]) + b_ref[...]


def _emb(x, w, b2):
    R = 1000
    return pl.pallas_call(
        _emb_body,
        grid=(N // R,),
        in_specs=[
            pl.BlockSpec((R, IN_DIM), lambda i: (i, 0)),
            pl.BlockSpec((HID, IN_DIM), lambda i: (0, 0)),
            pl.BlockSpec((1, HID), lambda i: (0, 0)),
        ],
        out_specs=pl.BlockSpec((R, HID), lambda i: (i, 0)),
        out_shape=jax.ShapeDtypeStruct((N, HID), jnp.float32),
    )(x, w, b2)


def _node_apply_body(h_ref, c0_ref, c1_ref, p0_ref, p1_ref, w_ref, b_ref,
                     o_ref):
    h = h_ref[...]
    deg = jnp.maximum(p0_ref[:, 0:1] + p1_ref[:, 0:1], 1.0)
    dinv = 1.0 / deg
    w = w_ref[...]
    z = (_dotT(h, w[:, 0:HID])
         + _dotT(c0_ref[...] * dinv, w[:, HID:HID + 128])
         + _dotT(c1_ref[...] * dinv, w[:, HID + 128:HID + 256])
         + b_ref[...])
    nrm = jnp.sqrt(jnp.sum(z * z, axis=1, keepdims=True))
    z = z / jnp.maximum(nrm, 1e-12)
    o_ref[...] = h + jnp.maximum(z, 0.0) * BN_SCALE


def _node_apply(h, c0, c1, p0, p1, w, b2):
    R = 1000
    return pl.pallas_call(
        _node_apply_body,
        grid=(N // R,),
        in_specs=[
            pl.BlockSpec((R, HID), lambda i: (i, 0)),
            pl.BlockSpec((R, 128), lambda i: (i, 0)),
            pl.BlockSpec((R, 128), lambda i: (i, 0)),
            pl.BlockSpec((R, 16), lambda i: (i, 0)),
            pl.BlockSpec((R, 16), lambda i: (i, 0)),
            pl.BlockSpec((HID, 2 * HID), lambda i: (0, 0)),
            pl.BlockSpec((1, HID), lambda i: (0, 0)),
        ],
        out_specs=pl.BlockSpec((R, HID), lambda i: (i, 0)),
        out_shape=jax.ShapeDtypeStruct((N, HID), jnp.float32),
    )(h, c0, c1, p0, p1, w, b2)


def _readout_body(h_ref, w0_ref, b0_ref, w1_ref, b1_ref, w2_ref, b2_ref,
                  o_ref):
    y = jnp.maximum(_dotT(h_ref[...], w0_ref[...]) + b0_ref[...], 0.0)
    y = jnp.maximum(_dotT(y, w1_ref[...]) + b1_ref[...], 0.0)
    o_ref[...] = _dotT(y, w2_ref[...]) + b2_ref[...]


def _readout(h, w0, b0, w1, b1, w2, b2):
    R = 1000
    return pl.pallas_call(
        _readout_body,
        grid=(N // R,),
        in_specs=[
            pl.BlockSpec((R, HID), lambda i: (i, 0)),
            pl.BlockSpec((128, HID), lambda i: (0, 0)),
            pl.BlockSpec((1, 128), lambda i: (0, 0)),
            pl.BlockSpec((64, 128), lambda i: (0, 0)),
            pl.BlockSpec((1, 64), lambda i: (0, 0)),
            pl.BlockSpec((2, 64), lambda i: (0, 0)),
            pl.BlockSpec((1, 2), lambda i: (0, 0)),
        ],
        out_specs=pl.BlockSpec((R, 2), lambda i: (i, 0)),
        out_shape=jax.ShapeDtypeStruct((N, 2), jnp.float32),
    )(h, w0, b0, w1, b1, w2, b2)


_MESH = plsc.VectorSubcoreMesh(core_axis_name="c", subcore_axis_name="s",
                               num_cores=_NSC, num_subcores=_NTILE)


@functools.partial(
    pl.kernel,
    out_type=jax.ShapeDtypeStruct((_NSC, N, 128), jnp.float32),
    mesh=_MESH,
    scratch_types=[
        pltpu.VMEM_SHARED((N, 128), jnp.float32),
        pltpu.VMEM((_K,), jnp.int32),
        pltpu.VMEM((_K,), jnp.int32),
        pltpu.VMEM((_K, 128), jnp.float32),
        pltpu.SemaphoreType.DMA,
    ],
)
def _segsum(h2_hbm, srcx_hbm, dst_hbm, zeros_hbm, out_hbm,
            acc, gidx, sidx, rows, sem):
    c = lax.axis_index("c")
    s = lax.axis_index("s")
    r0 = s * _ROWS_PER_TILE
    pltpu.sync_copy(zeros_hbm.at[pl.ds(r0, _ROWS_PER_TILE)],
                    acc.at[pl.ds(r0, _ROWS_PER_TILE)])
    plsc.subcore_barrier()

    def body(i, carry):
        ch = s + i * _NTILE

        @pl.when(ch < _NCH)
        def _():
            e0 = ch * _K
            pltpu.sync_copy(srcx_hbm.at[c, pl.ds(e0, _K)], gidx)
            pltpu.sync_copy(dst_hbm.at[pl.ds(e0, _K)], sidx)
            pltpu.async_copy(h2_hbm.at[gidx], rows, sem).wait()
            pltpu.sync_copy(rows, acc.at[sidx], add=True)

        return carry

    lax.fori_loop(0, (_NCH + _NTILE - 1) // _NTILE, body, 0)
    plsc.subcore_barrier()
    pltpu.sync_copy(acc.at[pl.ds(r0, _ROWS_PER_TILE)],
                    out_hbm.at[c, pl.ds(r0, _ROWS_PER_TILE)])


@functools.partial(
    pl.kernel,
    out_type=jax.ShapeDtypeStruct((_NSC, N, 16), jnp.float32),
    mesh=_MESH,
    scratch_types=[
        pltpu.VMEM_SHARED((N, 16), jnp.float32),
        pltpu.VMEM((_K,), jnp.int32),
        pltpu.VMEM((_K, 16), jnp.float32),
    ],
)
def _deg(dst_hbm, ones_hbm, zeros_hbm, out_hbm, acc, sidx, ones):
    c = lax.axis_index("c")
    s = lax.axis_index("s")
    r0 = s * _ROWS_PER_TILE
    pltpu.sync_copy(zeros_hbm.at[pl.ds(r0, _ROWS_PER_TILE)],
                    acc.at[pl.ds(r0, _ROWS_PER_TILE)])
    pltpu.sync_copy(ones_hbm, ones)
    plsc.subcore_barrier()
    half = _NCH // _NSC

    def body(i, carry):
        k = s + i * _NTILE

        @pl.when(k < half)
        def _():
            e0 = (c + _NSC * k) * _K
            pltpu.sync_copy(dst_hbm.at[pl.ds(e0, _K)], sidx)
            pltpu.sync_copy(ones, acc.at[sidx], add=True)

        return carry

    lax.fori_loop(0, (half + _NTILE - 1) // _NTILE, body, 0)
    plsc.subcore_barrier()
    pltpu.sync_copy(acc.at[pl.ds(r0, _ROWS_PER_TILE)],
                    out_hbm.at[c, pl.ds(r0, _ROWS_PER_TILE)])


def kernel(x, edge_index, W_emb, b_emb, W0, b0, W1, b1, W2, b2, W3, b3,
           Wm0, bm0, Wm1, bm1, Wm2, bm2):
    src = edge_index[0].astype(jnp.int32)
    dst = edge_index[1].astype(jnp.int32)
    srcx = jnp.stack([2 * src, 2 * src + 1])
    zeros128 = jnp.zeros((N, 128), jnp.float32)
    zeros16 = jnp.zeros((N, 16), jnp.float32)
    ones16 = jnp.zeros((_K, 16), jnp.float32).at[:, 0].set(1.0)

    h = _emb(x, W_emb, b_emb.reshape(1, -1))
    degp = _deg(dst, ones16, zeros16)
    for W, b in ((W0, b0), (W1, b1), (W2, b2), (W3, b3)):
        cs = _segsum(h.reshape(2 * N, 128), srcx, dst, zeros128)
        h = _node_apply(h, cs[0], cs[1], degp[0], degp[1], W,
                        b.reshape(1, -1))
    return _readout(h, Wm0, bm0.reshape(1, -1), Wm1, bm1.reshape(1, -1),
                    Wm2, bm2.reshape(1, -1))


# speedup vs baseline: 3.2805x; 3.2805x over previous
import functools

import jax
import jax.numpy as jnp
from jax import lax
from jax.experimental import pallas as pl
from jax.experimental.pallas import tpu as pltpu
from jax.experimental.pallas import tpu_sc as plsc

N = 10000
E = 160000
IN_DIM = 1024
HID = 256
BN_SCALE = 1.0 / (1.0 + 1e-5) ** 0.5

_NSC = 2
_NTILE = 16
_K = 128
_NCH = E // _K
_NPAD = 10240
_ROWS_PER_TILE = _NPAD // _NTILE

_PREC = jax.lax.Precision.HIGHEST


def _dotT(a, w):
    return lax.dot_general(a, w, (((1,), (1,)), ((), ())),
                           preferred_element_type=jnp.float32,
                           precision=_PREC)


def _emb_body(x_ref, w_ref, b_ref, o_ref):
    o_ref[...] = _dotT(x_ref[...], w_ref[...]) + b_ref[...]


def _emb(x, w, b2):
    R = 1000
    return pl.pallas_call(
        _emb_body,
        grid=(N // R,),
        in_specs=[
            pl.BlockSpec((R, IN_DIM), lambda i: (i, 0)),
            pl.BlockSpec((HID, IN_DIM), lambda i: (0, 0)),
            pl.BlockSpec((1, HID), lambda i: (0, 0)),
        ],
        out_specs=pl.BlockSpec((R, HID), lambda i: (i, 0)),
        out_shape=jax.ShapeDtypeStruct((N, HID), jnp.float32),
    )(x, w, b2)


def _node_apply_body(h_ref, c0_ref, c1_ref, p0_ref, p1_ref, w_ref, b_ref,
                     o_ref):
    h = h_ref[...]
    deg = jnp.maximum(p0_ref[:, 0:1] + p1_ref[:, 0:1], 1.0)
    dinv = 1.0 / deg
    w = w_ref[...]
    z = (_dotT(h, w[:, 0:HID])
         + _dotT(c0_ref[...] * dinv, w[:, HID:HID + 128])
         + _dotT(c1_ref[...] * dinv, w[:, HID + 128:HID + 256])
         + b_ref[...])
    nrm = jnp.sqrt(jnp.sum(z * z, axis=1, keepdims=True))
    z = z / jnp.maximum(nrm, 1e-12)
    o_ref[...] = h + jnp.maximum(z, 0.0) * BN_SCALE


def _node_apply(h, c0, c1, p0, p1, w, b2):
    R = 1000
    return pl.pallas_call(
        _node_apply_body,
        grid=(N // R,),
        in_specs=[
            pl.BlockSpec((R, HID), lambda i: (i, 0)),
            pl.BlockSpec((R, 128), lambda i: (i, 0)),
            pl.BlockSpec((R, 128), lambda i: (i, 0)),
            pl.BlockSpec((R, 128), lambda i: (i, 0)),
            pl.BlockSpec((R, 128), lambda i: (i, 0)),
            pl.BlockSpec((HID, 2 * HID), lambda i: (0, 0)),
            pl.BlockSpec((1, HID), lambda i: (0, 0)),
        ],
        out_specs=pl.BlockSpec((R, HID), lambda i: (i, 0)),
        out_shape=jax.ShapeDtypeStruct((N, HID), jnp.float32),
    )(h, c0, c1, p0, p1, w, b2)


def _readout_body(h_ref, w0_ref, b0_ref, w1_ref, b1_ref, w2_ref, b2_ref,
                  o_ref):
    y = jnp.maximum(_dotT(h_ref[...], w0_ref[...]) + b0_ref[...], 0.0)
    y = jnp.maximum(_dotT(y, w1_ref[...]) + b1_ref[...], 0.0)
    o_ref[...] = _dotT(y, w2_ref[...]) + b2_ref[...]


def _readout(h, w0, b0, w1, b1, w2, b2):
    R = 1000
    return pl.pallas_call(
        _readout_body,
        grid=(N // R,),
        in_specs=[
            pl.BlockSpec((R, HID), lambda i: (i, 0)),
            pl.BlockSpec((128, HID), lambda i: (0, 0)),
            pl.BlockSpec((1, 128), lambda i: (0, 0)),
            pl.BlockSpec((64, 128), lambda i: (0, 0)),
            pl.BlockSpec((1, 64), lambda i: (0, 0)),
            pl.BlockSpec((2, 64), lambda i: (0, 0)),
            pl.BlockSpec((1, 2), lambda i: (0, 0)),
        ],
        out_specs=pl.BlockSpec((R, 2), lambda i: (i, 0)),
        out_shape=jax.ShapeDtypeStruct((N, 2), jnp.float32),
    )(h, w0, b0, w1, b1, w2, b2)


def _sc_mesh():
    return plsc.VectorSubcoreMesh(core_axis_name="c", subcore_axis_name="s",
                                  num_cores=_NSC, num_subcores=_NTILE)


@functools.cache
def _make_segsum():
    return functools.partial(
        pl.kernel,
        out_type=jax.ShapeDtypeStruct((_NSC, _NPAD, 128), jnp.float32),
        mesh=_sc_mesh(),
        scratch_types=[
            pltpu.VMEM_SHARED((_NPAD, 128), jnp.float32),
            pltpu.VMEM((_K,), jnp.int32),
            pltpu.VMEM((_K,), jnp.int32),
            pltpu.VMEM((_K, 128), jnp.float32),
            pltpu.SemaphoreType.DMA,
        ],
    )(_segsum_body)


def _segsum(h2, srcx, dst, zeros):
    return _make_segsum()(h2, srcx, dst, zeros)


def _segsum_body(h2_hbm, srcx_hbm, dst_hbm, zeros_hbm, out_hbm,
                 acc, gidx, sidx, rows, sem):
    c = lax.axis_index("c")
    s = lax.axis_index("s")
    r0 = s * _ROWS_PER_TILE
    pltpu.sync_copy(zeros_hbm.at[pl.ds(r0, _ROWS_PER_TILE)],
                    acc.at[pl.ds(r0, _ROWS_PER_TILE)])
    plsc.subcore_barrier()

    def body(i, carry):
        ch = s + i * _NTILE

        @pl.when(ch < _NCH)
        def _():
            e0 = ch * _K
            pltpu.sync_copy(srcx_hbm.at[pl.ds((2 * ch + c) * _K, _K)], gidx)
            pltpu.sync_copy(dst_hbm.at[pl.ds(e0, _K)], sidx)
            pltpu.async_copy(h2_hbm.at[gidx], rows, sem).wait()
            pltpu.sync_copy(rows, acc.at[sidx], add=True)

        return carry

    lax.fori_loop(0, (_NCH + _NTILE - 1) // _NTILE, body, 0)
    plsc.subcore_barrier()
    pltpu.sync_copy(acc.at[pl.ds(r0, _ROWS_PER_TILE)],
                    out_hbm.at[c, pl.ds(r0, _ROWS_PER_TILE)])


@functools.cache
def _make_deg():
    return functools.partial(
        pl.kernel,
        out_type=jax.ShapeDtypeStruct((_NSC, _NPAD, 128), jnp.float32),
        mesh=_sc_mesh(),
        scratch_types=[
            pltpu.VMEM_SHARED((_NPAD, 128), jnp.float32),
            pltpu.VMEM((_K,), jnp.int32),
            pltpu.VMEM((_K, 128), jnp.float32),
        ],
    )(_deg_body)


def _deg(dst, ones, zeros):
    return _make_deg()(dst, ones, zeros)


def _deg_body(dst_hbm, ones_hbm, zeros_hbm, out_hbm, acc, sidx, ones):
    c = lax.axis_index("c")
    s = lax.axis_index("s")
    r0 = s * _ROWS_PER_TILE
    pltpu.sync_copy(zeros_hbm.at[pl.ds(r0, _ROWS_PER_TILE)],
                    acc.at[pl.ds(r0, _ROWS_PER_TILE)])
    pltpu.sync_copy(ones_hbm, ones)
    plsc.subcore_barrier()
    half = _NCH // _NSC

    def body(i, carry):
        k = s + i * _NTILE

        @pl.when(k < half)
        def _():
            e0 = (c + _NSC * k) * _K
            pltpu.sync_copy(dst_hbm.at[pl.ds(e0, _K)], sidx)
            pltpu.sync_copy(ones, acc.at[sidx], add=True)

        return carry

    lax.fori_loop(0, (half + _NTILE - 1) // _NTILE, body, 0)
    plsc.subcore_barrier()
    pltpu.sync_copy(acc.at[pl.ds(r0, _ROWS_PER_TILE)],
                    out_hbm.at[c, pl.ds(r0, _ROWS_PER_TILE)])


def kernel(x, edge_index, W_emb, b_emb, W0, b0, W1, b1, W2, b2, W3, b3,
           Wm0, bm0, Wm1, bm1, Wm2, bm2):
    src = edge_index[0].astype(jnp.int32)
    dst = edge_index[1].astype(jnp.int32)
    s2 = (2 * src).reshape(_NCH, 1, _K)
    srcx = jnp.concatenate([s2, s2 + 1], axis=1).reshape(-1)
    zeros128 = jnp.zeros((_NPAD, 128), jnp.float32)
    ones128 = jnp.zeros((_K, 128), jnp.float32).at[:, 0].set(1.0)

    h = _emb(x, W_emb, b_emb.reshape(1, -1))
    degp = _deg(dst, ones128, zeros128)
    p0, p1 = degp[0, :N], degp[1, :N]
    for W, b in ((W0, b0), (W1, b1), (W2, b2), (W3, b3)):
        cs = _segsum(h.reshape(2 * N, 128), srcx, dst, zeros128)
        h = _node_apply(h, cs[0, :N], cs[1, :N], p0, p1, W,
                        b.reshape(1, -1))
    return _readout(h, Wm0, bm0.reshape(1, -1), Wm1, bm1.reshape(1, -1),
                    Wm2, bm2.reshape(1, -1))
